# baseline (speedup 1.0000x reference)
import functools

import jax
import jax.numpy as jnp
from jax import lax
from jax.experimental import pallas as pl
from jax.experimental.pallas import tpu as pltpu
from jax.experimental.pallas import tpu_sc as plsc

_DEPTH = 8192
_FEAT = 4096
_DECAY = 0.9
_LANES = 16
_NC, _NS = 2, 16
_NW = _NC * _NS
_ROWS_W = _DEPTH // _NW
_CHUNK = 8
_NCHUNK = _ROWS_W // _CHUNK


def _fixup_body(w_ref, p_ref, i_ref, t_ref, blended_ref, out_ref):
    w = jnp.mean(w_ref[...], axis=0)
    p = jnp.mean(p_ref[...], axis=0)
    it = jnp.mean(i_ref[...], axis=0)
    nm = jnp.concatenate([w, p, it], axis=-1)
    out_ref[...] = blended_ref[...]
    out_ref[0:1, :] = t_ref[0:1, :] * _DECAY + nm[None, :] * (1.0 - _DECAY)


def _fixup_row0(world_embed, psi, intent, T, blended):
    return pl.pallas_call(
        _fixup_body,
        grid=(1,),
        in_specs=[
            pl.BlockSpec(world_embed.shape, lambda i: (0, 0)),
            pl.BlockSpec(psi.shape, lambda i: (0, 0)),
            pl.BlockSpec(intent.shape, lambda i: (0, 0)),
            pl.BlockSpec((8, _FEAT), lambda i: (0, 0)),
            pl.BlockSpec((8, _FEAT), lambda i: (0, 0)),
        ],
        out_specs=pl.BlockSpec((8, _FEAT), lambda i: (0, 0)),
        out_shape=jax.ShapeDtypeStruct((_DEPTH, _FEAT), jnp.float32),
        input_output_aliases={4: 0},
    )(world_embed, psi, intent, T, blended)


def _blend_call(T):
    mesh = plsc.VectorSubcoreMesh(core_axis_name="c", subcore_axis_name="s")

    @functools.partial(
        pl.kernel,
        out_type=jax.ShapeDtypeStruct((_DEPTH, _FEAT), jnp.float32),
        mesh=mesh,
        scratch_types=[
            pltpu.VMEM((_CHUNK, _FEAT), jnp.float32),
            pltpu.VMEM((_CHUNK, _FEAT), jnp.float32),
            pltpu.VMEM((_CHUNK, _FEAT), jnp.float32),
            pltpu.VMEM((1, _FEAT), jnp.float32),
            pltpu.SemaphoreType.DMA,
            pltpu.SemaphoreType.DMA,
            pltpu.SemaphoreType.DMA,
            pltpu.SemaphoreType.DMA,
            pltpu.SemaphoreType.DMA,
            pltpu.SemaphoreType.DMA,
            pltpu.SemaphoreType.DMA,
        ],
    )
    def blend(t_hbm, out_hbm, buf0, buf1, buf2, halo_save,
              isem0, isem1, isem2, osem0, osem1, osem2, hsem):
        cid = lax.axis_index("c")
        sid = lax.axis_index("s")
        wid = sid * _NC + cid
        base = wid * _ROWS_W
        bufs = (buf0, buf1, buf2)
        isems = (isem0, isem1, isem2)
        osems = (osem0, osem1, osem2)

        def wait_in(buf, sem):
            pltpu.make_async_copy(t_hbm.at[pl.ds(0, _CHUNK)], buf, sem).wait()

        def wait_out(buf, sem):
            pltpu.make_async_copy(buf, out_hbm.at[pl.ds(0, _CHUNK)], sem).wait()

        def compute(buf):
            @plsc.parallel_loop(0, _FEAT // _LANES, unroll=8)
            def _strip(j):
                off = j * _LANES
                carry = halo_save[0, pl.ds(off, _LANES)]
                for k in range(_CHUNK):
                    a = buf[k, pl.ds(off, _LANES)]
                    buf[k, pl.ds(off, _LANES)] = a + (carry - a) * (1.0 - _DECAY)
                    carry = a
                halo_save[0, pl.ds(off, _LANES)] = carry

        halo_off = jnp.where(wid == 0, 0, base - 8)
        pltpu.async_copy(t_hbm.at[pl.ds(halo_off, 8)], buf2, hsem)
        pltpu.async_copy(t_hbm.at[pl.ds(base, _CHUNK)], buf0, isem0)
        pltpu.async_copy(t_hbm.at[pl.ds(base + _CHUNK, _CHUNK)], buf1, isem1)

        pltpu.make_async_copy(t_hbm.at[pl.ds(0, 8)], buf2, hsem).wait()

        @plsc.parallel_loop(0, _FEAT // _LANES, unroll=8)
        def _seed(j):
            off = j * _LANES
            halo_save[0, pl.ds(off, _LANES)] = buf2[7, pl.ds(off, _LANES)]

        def process(c, b):
            buf, isem, osem = bufs[b], isems[b], osems[b]
            b2 = (b + 2) % 3
            nbuf, nisem, nosem = bufs[b2], isems[b2], osems[b2]

            @pl.when(c + 2 < _NCHUNK)
            def _():
                @pl.when(c >= 1)
                def _():
                    wait_out(nbuf, nosem)

                pltpu.async_copy(
                    t_hbm.at[pl.ds(base + (c + 2) * _CHUNK, _CHUNK)], nbuf, nisem
                )

            wait_in(buf, isem)
            compute(buf)
            pltpu.async_copy(
                buf, out_hbm.at[pl.ds(base + c * _CHUNK, _CHUNK)], osem
            )

        def triple_body(g, _):
            process(3 * g, 0)
            process(3 * g + 1, 1)
            process(3 * g + 2, 2)
            return 0

        lax.fori_loop(0, (_NCHUNK - 2) // 3, triple_body, 0)
        process(_NCHUNK - 2, (_NCHUNK - 2) % 3)
        process(_NCHUNK - 1, (_NCHUNK - 1) % 3)
        wait_out(buf0, osem0)
        wait_out(buf1, osem1)
        wait_out(buf2, osem2)

    return blend(T)


def kernel(world_embed, psi, intent, T):
    blended = _blend_call(T)
    return _fixup_row0(world_embed, psi, intent, T, blended)


# speedup vs baseline: 1.0260x; 1.0260x over previous
import functools

import jax
import jax.numpy as jnp
from jax import lax
from jax.experimental import pallas as pl
from jax.experimental.pallas import tpu as pltpu
from jax.experimental.pallas import tpu_sc as plsc

_DEPTH = 8192
_FEAT = 4096
_DECAY = 0.9
_LANES = 16
_NC, _NS = 2, 16
_NW = _NC * _NS
_ROWS_W = _DEPTH // _NW
_CHUNK = 8
_NCHUNK = _ROWS_W // _CHUNK


def _mean_body(w_ref, p_ref, i_ref, out_ref):
    w = jnp.mean(w_ref[...], axis=0)
    p = jnp.mean(p_ref[...], axis=0)
    it = jnp.mean(i_ref[...], axis=0)
    nm = jnp.concatenate([w, p, it], axis=-1)
    out_ref[...] = jnp.broadcast_to(nm[None, :], (8, _FEAT))


def _new_memory(world_embed, psi, intent):
    return pl.pallas_call(
        _mean_body,
        out_shape=jax.ShapeDtypeStruct((8, _FEAT), jnp.float32),
    )(world_embed, psi, intent)


def _blend_call(nm, T):
    mesh = plsc.VectorSubcoreMesh(core_axis_name="c", subcore_axis_name="s")

    @functools.partial(
        pl.kernel,
        out_type=jax.ShapeDtypeStruct((_DEPTH, _FEAT), jnp.float32),
        mesh=mesh,
        scratch_types=[
            pltpu.VMEM((_CHUNK, _FEAT), jnp.float32),
            pltpu.VMEM((_CHUNK, _FEAT), jnp.float32),
            pltpu.VMEM((8, _FEAT), jnp.float32),
            pltpu.VMEM((1, _FEAT), jnp.float32),
            pltpu.SemaphoreType.DMA,
            pltpu.SemaphoreType.DMA,
            pltpu.SemaphoreType.DMA,
            pltpu.SemaphoreType.DMA,
            pltpu.SemaphoreType.DMA,
        ],
    )
    def blend(nm_hbm, t_hbm, out_hbm, buf0, buf1, halo_buf, halo_save,
              isem0, isem1, osem0, osem1, hsem):
        cid = lax.axis_index("c")
        sid = lax.axis_index("s")
        wid = sid * _NC + cid
        base = wid * _ROWS_W
        bufs = (buf0, buf1)
        isems = (isem0, isem1)
        osems = (osem0, osem1)

        def wait_in(buf, sem):
            pltpu.make_async_copy(t_hbm.at[pl.ds(0, _CHUNK)], buf, sem).wait()

        def wait_out(buf, sem):
            pltpu.make_async_copy(buf, out_hbm.at[pl.ds(0, _CHUNK)], sem).wait()

        def compute(buf):
            @plsc.parallel_loop(0, _FEAT // _LANES, unroll=8)
            def _strip(j):
                off = j * _LANES
                carry = halo_save[0, pl.ds(off, _LANES)]
                for k in range(_CHUNK):
                    a = buf[k, pl.ds(off, _LANES)]
                    buf[k, pl.ds(off, _LANES)] = a * _DECAY + carry * (1.0 - _DECAY)
                    carry = a
                halo_save[0, pl.ds(off, _LANES)] = carry

        @pl.when(wid == 0)
        def _():
            pltpu.async_copy(nm_hbm, halo_buf, hsem)

        @pl.when(wid > 0)
        def _():
            pltpu.async_copy(t_hbm.at[pl.ds(base - 8, 8)], halo_buf, hsem)

        pltpu.async_copy(t_hbm.at[pl.ds(base, _CHUNK)], buf0, isem0)

        pltpu.make_async_copy(nm_hbm, halo_buf, hsem).wait()

        @plsc.parallel_loop(0, _FEAT // _LANES, unroll=8)
        def _seed(j):
            off = j * _LANES
            halo_save[0, pl.ds(off, _LANES)] = halo_buf[7, pl.ds(off, _LANES)]

        def process(c, b):
            buf, isem, osem = bufs[b], isems[b], osems[b]
            nbuf, nisem, nosem = bufs[1 - b], isems[1 - b], osems[1 - b]

            @pl.when(c + 1 < _NCHUNK)
            def _():
                @pl.when(c >= 1)
                def _():
                    wait_out(nbuf, nosem)

                pltpu.async_copy(
                    t_hbm.at[pl.ds(base + (c + 1) * _CHUNK, _CHUNK)], nbuf, nisem
                )

            wait_in(buf, isem)
            compute(buf)
            pltpu.async_copy(
                buf, out_hbm.at[pl.ds(base + c * _CHUNK, _CHUNK)], osem
            )

        def pair_body(g, _):
            process(2 * g, 0)
            process(2 * g + 1, 1)
            return 0

        lax.fori_loop(0, _NCHUNK // 2, pair_body, 0)
        wait_out(buf0, osem0)
        wait_out(buf1, osem1)

    return blend(nm, T)


def kernel(world_embed, psi, intent, T):
    nm = _new_memory(world_embed, psi, intent)
    return _blend_call(nm, T)
